# baseline (speedup 1.0000x reference)
import jax
import jax.numpy as jnp
from jax import lax
from jax.experimental import pallas as pl
from jax.experimental.pallas import tpu as pltpu
from jax.experimental.pallas import tpu_sc as plsc

NC = 2
NS = 16
L = 16
NW = NC * NS

B = 16384
V = 1000000
D = 64
LANES = 128
NSLAB = (V + LANES - 1) // LANES
SPT = (NSLAB + NS - 1) // NS
BATCH = 128
DUMP = B
BPW = B // NW
PAD_HPK = 16400


def _p1_body(e1_hbm, e2_hbm, pk1_hbm, pk2_hbm, hpk1_hbm, hpk2_hbm, ts_hbm,
             g1_hbm, g2_hbm,
             pk_v, hpk_v, ts_v, slab_v, row_v, dest_v,
             sem_a, sem_b, sem_s):
    c = lax.axis_index("c")
    s = lax.axis_index("s")

    pltpu.sync_copy(ts_hbm, ts_v)

    def dest_reset():
        for k in range(BATCH // L):
            dest_v[pl.ds(k * L, L)] = jnp.full((L,), DUMP, jnp.int32)

    lane16 = lax.iota(jnp.int32, L)

    def run(e_hbm, pk_hbm, hpk_hbm, g_hbm, toff):
        pltpu.sync_copy(pk_hbm, pk_v)
        pltpu.sync_copy(hpk_hbm, hpk_v)
        dest_reset()

        tvec = ts_v[pl.ds(toff + s, L)]
        d_lo = tvec[0]
        d_hi = tvec[1]

        def slab_dma(dd, buf, sem):
            slab = hpk_v[pl.ds(dd, L)][0] & 8191
            return pltpu.async_copy(
                e_hbm.at[:, pl.ds(slab * LANES, LANES)],
                slab_v.at[buf], sem)

        @pl.when(d_lo < d_hi)
        def _():
            slab_dma(d_lo, 0, sem_a)

        def fire_scatter():
            pltpu.async_copy(row_v, g_hbm.at[dest_v], sem_s).wait()
            dest_reset()

        def dd_body(dd, m):
            k = (dd - d_lo) % 2

            @pl.when((k == 0) & (dd + 1 < d_hi))
            def _():
                slab_dma(dd + 1, 1, sem_b)

            @pl.when((k == 1) & (dd + 1 < d_hi))
            def _():
                slab_dma(dd + 1, 0, sem_a)

            @pl.when(k == 0)
            def _():
                pltpu.make_async_copy(
                    e_hbm.at[:, pl.ds(0, LANES)], slab_v.at[0], sem_a).wait()

            @pl.when(k == 1)
            def _():
                pltpu.make_async_copy(
                    e_hbm.at[:, pl.ds(0, LANES)], slab_v.at[1], sem_b).wait()

            hvec = hpk_v[pl.ds(dd, L)]
            hs = hvec[0] >> 13
            he = hvec[1] >> 13
            kk = jnp.full((L,), k, jnp.int32)

            def hit_body(p, m):
                pk = pk_v[pl.ds(p, L)][0]
                ln = jnp.full((L,), pk & 127, jnp.int32)
                dest = pk >> 7
                for j in range(D // L):
                    v = plsc.load_gather(
                        slab_v, [kk, j * L + lane16, ln])
                    row_v[m, pl.ds(j * L, L)] = v
                plsc.store_scatter(dest_v, [jnp.full((L,), m, jnp.int32)],
                                   jnp.full((L,), dest, jnp.int32),
                                   mask=lane16 == 0)
                m = m + 1

                @pl.when(m == BATCH)
                def _():
                    fire_scatter()

                return jnp.where(m == BATCH, 0, m)

            return lax.fori_loop(hs, he, hit_body, m)

        m = lax.fori_loop(d_lo, d_hi, dd_body, 0)

        @pl.when(m > 0)
        def _():
            fire_scatter()

    @pl.when(c == 0)
    def _():
        run(e1_hbm, pk1_hbm, hpk1_hbm, g1_hbm, 0)

    @pl.when(c == 1)
    def _():
        run(e2_hbm, pk2_hbm, hpk2_hbm, g2_hbm, NS + 1)


def _p2_body(g1_hbm, g2_hbm, out_hbm, c1_v, c2_v, out_v):
    wid = lax.axis_index("s") * NC + lax.axis_index("c")
    base = wid * BPW
    lane16 = lax.iota(jnp.int32, L)

    for j in range(BPW // BATCH):
        pltpu.sync_copy(g1_hbm.at[pl.ds(base + j * BATCH, BATCH)], c1_v)
        pltpu.sync_copy(g2_hbm.at[pl.ds(base + j * BATCH, BATCH)], c2_v)

        def group_body(g, _):
            sums = jnp.zeros((L,), jnp.float32)
            for k in range(L):
                r = g * L + k
                acc = jnp.zeros((L,), jnp.float32)
                for q in range(D // L):
                    a = c1_v[r, pl.ds(q * L, L)]
                    b = c2_v[r, pl.ds(q * L, L)]
                    acc = acc + a * b
                sums = jnp.where(lane16 == k, jnp.sum(acc), sums)
            y = 1.0 / (1.0 + jnp.exp(-sums))
            out_v[pl.ds(j * BATCH + g * L, L)] = y
            return 0

        lax.fori_loop(0, BATCH // L, group_body, 0)

    pltpu.sync_copy(out_v, out_hbm.at[pl.ds(base, BPW)])


def _prep(idx):
    perm = jnp.argsort(idx).astype(jnp.int32)
    sr = idx[perm]
    keys = sr >> 7
    flags = jnp.concatenate([jnp.ones((1,), jnp.bool_), keys[1:] != keys[:-1]])
    dpos = jnp.cumsum(flags.astype(jnp.int32)) - 1
    hstart = jnp.searchsorted(dpos, jnp.arange(PAD_HPK, dtype=jnp.int32)
                              ).astype(jnp.int32)
    keys_pad = jnp.concatenate([keys, jnp.full((1,), NSLAB, jnp.int32)])
    dlist = keys_pad[jnp.minimum(hstart, B)]
    hpk = (hstart << 13) | dlist
    pk = jnp.concatenate([(perm << 7) | (sr & 127),
                          jnp.zeros((PAD_HPK - B,), jnp.int32)])
    bnd = jnp.minimum(jnp.arange(NS + 1, dtype=jnp.int32) * SPT, NSLAB)
    td = jnp.searchsorted(dlist[:B], bnd).astype(jnp.int32)
    return pk, hpk, td


@jax.jit
def _run(X, emb1, emb2):
    wcf = X[:, 0]
    wof = X[:, 1]
    pk1, hpk1, td1 = _prep(wcf)
    pk2, hpk2, td2 = _prep(wof)
    ts = jnp.concatenate([td1, td2,
                          jnp.zeros((64 - 2 * (NS + 1),), jnp.int32)])

    mesh = plsc.VectorSubcoreMesh(core_axis_name="c", subcore_axis_name="s",
                                  num_cores=NC, num_subcores=NS)
    g1, g2 = pl.kernel(
        _p1_body,
        out_type=(jax.ShapeDtypeStruct((B + 1, LANES), jnp.float32),
                  jax.ShapeDtypeStruct((B + 1, LANES), jnp.float32)),
        mesh=mesh,
        scratch_types=[
            pltpu.VMEM((PAD_HPK,), jnp.int32),
            pltpu.VMEM((PAD_HPK,), jnp.int32),
            pltpu.VMEM((64,), jnp.int32),
            pltpu.VMEM((2, D, LANES), jnp.float32),
            pltpu.VMEM((BATCH, LANES), jnp.float32),
            pltpu.VMEM((BATCH,), jnp.int32),
            pltpu.SemaphoreType.DMA,
            pltpu.SemaphoreType.DMA,
            pltpu.SemaphoreType.DMA,
        ],
        compiler_params=pltpu.CompilerParams(needs_layout_passes=False),
    )(emb1.T, emb2.T, pk1, pk2, hpk1, hpk2, ts)

    return pl.kernel(
        _p2_body,
        out_type=jax.ShapeDtypeStruct((B,), jnp.float32),
        mesh=mesh,
        scratch_types=[
            pltpu.VMEM((BATCH, LANES), jnp.float32),
            pltpu.VMEM((BATCH, LANES), jnp.float32),
            pltpu.VMEM((BPW,), jnp.float32),
        ],
        compiler_params=pltpu.CompilerParams(needs_layout_passes=False),
    )(g1, g2)


def kernel(X, emb1, emb2):
    return _run(X, emb1, emb2)


# speedup vs baseline: 4.1676x; 4.1676x over previous
import jax
import jax.numpy as jnp
from jax import lax
from jax.experimental import pallas as pl
from jax.experimental.pallas import tpu as pltpu
from jax.experimental.pallas import tpu_sc as plsc

NC = 2
NS = 16
L = 16
NW = NC * NS

B = 16384
V = 1000000
D = 64
LANES = 128
NSLAB = (V + LANES - 1) // LANES
SPT = (NSLAB + NS - 1) // NS
BATCH = 128
DUMP = B
BPW = B // NW
PAD_HPK = 16400


def _p1_body(e1_hbm, e2_hbm, pk1_hbm, pk2_hbm, hpk1_hbm, hpk2_hbm, ts_hbm,
             g1_hbm, g2_hbm,
             pk_v, hpk_v, ts_v, slab_v, row_v, dest_v,
             sem_a, sem_b, sem_s):
    c = lax.axis_index("c")
    s = lax.axis_index("s")

    pltpu.sync_copy(ts_hbm, ts_v)

    def dest_reset():
        for k in range(BATCH // L):
            dest_v[pl.ds(k * L, L)] = jnp.full((L,), DUMP, jnp.int32)

    lane16 = lax.iota(jnp.int32, L)

    def run(e_hbm, pk_hbm, hpk_hbm, g_hbm, toff):
        pltpu.sync_copy(pk_hbm, pk_v)
        pltpu.sync_copy(hpk_hbm, hpk_v)
        dest_reset()

        tvec = ts_v[pl.ds(toff + s, L)]
        d_lo = tvec[0]
        d_hi = tvec[1]

        def slab_dma(dd, buf, sem):
            slab = hpk_v[pl.ds(dd, L)][0] & 8191
            return pltpu.async_copy(
                e_hbm.at[:, pl.ds(slab * LANES, LANES)],
                slab_v.at[buf], sem)

        @pl.when(d_lo < d_hi)
        def _():
            slab_dma(d_lo, 0, sem_a)

        def fire_scatter():
            pltpu.async_copy(row_v, g_hbm.at[dest_v], sem_s).wait()
            dest_reset()

        def dd_body(dd, m):
            k = (dd - d_lo) % 2

            @pl.when((k == 0) & (dd + 1 < d_hi))
            def _():
                slab_dma(dd + 1, 1, sem_b)

            @pl.when((k == 1) & (dd + 1 < d_hi))
            def _():
                slab_dma(dd + 1, 0, sem_a)

            @pl.when(k == 0)
            def _():
                pltpu.make_async_copy(
                    e_hbm.at[:, pl.ds(0, LANES)], slab_v.at[0], sem_a).wait()

            @pl.when(k == 1)
            def _():
                pltpu.make_async_copy(
                    e_hbm.at[:, pl.ds(0, LANES)], slab_v.at[1], sem_b).wait()

            hvec = hpk_v[pl.ds(dd, L)]
            hs = hvec[0] >> 13
            he = hvec[1] >> 13
            kk = jnp.full((L,), k, jnp.int32)

            def hit_body(p, m):
                pk = pk_v[pl.ds(p, L)][0]
                ln = jnp.full((L,), pk & 127, jnp.int32)
                dest = pk >> 7
                for j in range(D // L):
                    v = plsc.load_gather(
                        slab_v, [kk, j * L + lane16, ln])
                    row_v[m, pl.ds(j * L, L)] = v
                plsc.store_scatter(dest_v, [jnp.full((L,), m, jnp.int32)],
                                   jnp.full((L,), dest, jnp.int32),
                                   mask=lane16 == 0)
                m = m + 1

                @pl.when(m == BATCH)
                def _():
                    fire_scatter()

                return jnp.where(m == BATCH, 0, m)

            return lax.fori_loop(hs, he, hit_body, m)

        m = lax.fori_loop(d_lo, d_hi, dd_body, 0)

        @pl.when(m > 0)
        def _():
            fire_scatter()

    @pl.when(c == 0)
    def _():
        run(e1_hbm, pk1_hbm, hpk1_hbm, g1_hbm, 0)

    @pl.when(c == 1)
    def _():
        run(e2_hbm, pk2_hbm, hpk2_hbm, g2_hbm, NS + 1)


def _p2_body(g1_hbm, g2_hbm, out_hbm, c1_v, c2_v, out_v):
    wid = lax.axis_index("s") * NC + lax.axis_index("c")
    base = wid * BPW
    lane16 = lax.iota(jnp.int32, L)

    for j in range(BPW // BATCH):
        pltpu.sync_copy(g1_hbm.at[pl.ds(base + j * BATCH, BATCH)], c1_v)
        pltpu.sync_copy(g2_hbm.at[pl.ds(base + j * BATCH, BATCH)], c2_v)

        def group_body(g, _):
            sums = jnp.zeros((L,), jnp.float32)
            for k in range(L):
                r = g * L + k
                acc = jnp.zeros((L,), jnp.float32)
                for q in range(D // L):
                    a = c1_v[r, pl.ds(q * L, L)]
                    b = c2_v[r, pl.ds(q * L, L)]
                    acc = acc + a * b
                sums = jnp.where(lane16 == k, jnp.sum(acc), sums)
            y = 1.0 / (1.0 + jnp.exp(-sums))
            out_v[pl.ds(j * BATCH + g * L, L)] = y
            return 0

        lax.fori_loop(0, BATCH // L, group_body, 0)

    pltpu.sync_copy(out_v, out_hbm.at[pl.ds(base, BPW)])


def _prep(idx):
    perm = jnp.argsort(idx).astype(jnp.int32)
    sr = idx[perm]
    keys = sr >> 7
    pos = jnp.arange(B, dtype=jnp.int32)
    flags = jnp.concatenate([jnp.ones((1,), jnp.bool_), keys[1:] != keys[:-1]])
    hstart = jnp.concatenate([jnp.sort(jnp.where(flags, pos, B)),
                              jnp.full((PAD_HPK - B,), B, jnp.int32)])
    keys_pad = jnp.concatenate([keys, jnp.full((1,), NSLAB, jnp.int32)])
    dlist = keys_pad[jnp.minimum(hstart, B)]
    hpk = (hstart << 13) | dlist
    pk = jnp.concatenate([(perm << 7) | (sr & 127),
                          jnp.zeros((PAD_HPK - B,), jnp.int32)])
    bnd = jnp.minimum(jnp.arange(NS + 1, dtype=jnp.int32) * SPT, NSLAB)
    td = jnp.searchsorted(dlist[:B], bnd).astype(jnp.int32)
    return pk, hpk, td


@jax.jit
def _run(X, emb1, emb2):
    wcf = X[:, 0]
    wof = X[:, 1]
    pk1, hpk1, td1 = _prep(wcf)
    pk2, hpk2, td2 = _prep(wof)
    ts = jnp.concatenate([td1, td2,
                          jnp.zeros((64 - 2 * (NS + 1),), jnp.int32)])

    mesh = plsc.VectorSubcoreMesh(core_axis_name="c", subcore_axis_name="s",
                                  num_cores=NC, num_subcores=NS)
    g1, g2 = pl.kernel(
        _p1_body,
        out_type=(jax.ShapeDtypeStruct((B + 1, LANES), jnp.float32),
                  jax.ShapeDtypeStruct((B + 1, LANES), jnp.float32)),
        mesh=mesh,
        scratch_types=[
            pltpu.VMEM((PAD_HPK,), jnp.int32),
            pltpu.VMEM((PAD_HPK,), jnp.int32),
            pltpu.VMEM((64,), jnp.int32),
            pltpu.VMEM((2, D, LANES), jnp.float32),
            pltpu.VMEM((BATCH, LANES), jnp.float32),
            pltpu.VMEM((BATCH,), jnp.int32),
            pltpu.SemaphoreType.DMA,
            pltpu.SemaphoreType.DMA,
            pltpu.SemaphoreType.DMA,
        ],
        compiler_params=pltpu.CompilerParams(needs_layout_passes=False),
    )(emb1.T, emb2.T, pk1, pk2, hpk1, hpk2, ts)

    return pl.kernel(
        _p2_body,
        out_type=jax.ShapeDtypeStruct((B,), jnp.float32),
        mesh=mesh,
        scratch_types=[
            pltpu.VMEM((BATCH, LANES), jnp.float32),
            pltpu.VMEM((BATCH, LANES), jnp.float32),
            pltpu.VMEM((BPW,), jnp.float32),
        ],
        compiler_params=pltpu.CompilerParams(needs_layout_passes=False),
    )(g1, g2)


def kernel(X, emb1, emb2):
    return _run(X, emb1, emb2)


# speedup vs baseline: 4.2948x; 1.0305x over previous
import jax
import jax.numpy as jnp
from jax import lax
from jax.experimental import pallas as pl
from jax.experimental.pallas import tpu as pltpu
from jax.experimental.pallas import tpu_sc as plsc

NC = 2
NS = 16
L = 16
NW = NC * NS

B = 16384
V = 1000000
D = 64
LANES = 128
SS = 512
NSLAB = (V + SS - 1) // SS
SPT = (NSLAB + NS - 1) // NS
MAXB13 = (1000064 - SS) // LANES
BATCH = 128
DUMP = B
BPW = B // NW
PAD_HPK = 16400


def _p1_body(e1_hbm, e2_hbm, pk1_hbm, pk2_hbm, hpk1_hbm, hpk2_hbm, ts_hbm,
             g1_hbm, g2_hbm,
             pk_v, hpk_v, ts_v, slab_v, row_v, dest_v,
             sem_a, sem_b, sem_s):
    c = lax.axis_index("c")
    s = lax.axis_index("s")

    pltpu.sync_copy(ts_hbm, ts_v)

    def dest_reset():
        for k in range(BATCH // L):
            dest_v[pl.ds(k * L, L)] = jnp.full((L,), DUMP, jnp.int32)

    lane16 = lax.iota(jnp.int32, L)

    def run(e_hbm, pk_hbm, hpk_hbm, g_hbm, toff):
        pltpu.sync_copy(pk_hbm, pk_v)
        pltpu.sync_copy(hpk_hbm, hpk_v)
        dest_reset()

        tvec = ts_v[pl.ds(toff + s, L)]
        d_lo = tvec[0]
        d_hi = tvec[1]

        def slab_dma(dd, buf, sem):
            base13 = hpk_v[pl.ds(dd, L)][0] & 8191
            return pltpu.async_copy(
                e_hbm.at[:, pl.ds(base13 * LANES, SS)],
                slab_v.at[buf], sem)

        @pl.when(d_lo < d_hi)
        def _():
            slab_dma(d_lo, 0, sem_a)

        def fire_scatter():
            pltpu.async_copy(row_v, g_hbm.at[dest_v], sem_s).wait()
            dest_reset()

        def dd_body(dd, m):
            k = (dd - d_lo) % 2

            @pl.when((k == 0) & (dd + 1 < d_hi))
            def _():
                slab_dma(dd + 1, 1, sem_b)

            @pl.when((k == 1) & (dd + 1 < d_hi))
            def _():
                slab_dma(dd + 1, 0, sem_a)

            @pl.when(k == 0)
            def _():
                pltpu.make_async_copy(
                    e_hbm.at[:, pl.ds(0, SS)], slab_v.at[0], sem_a).wait()

            @pl.when(k == 1)
            def _():
                pltpu.make_async_copy(
                    e_hbm.at[:, pl.ds(0, SS)], slab_v.at[1], sem_b).wait()

            hvec = hpk_v[pl.ds(dd, L)]
            hs = hvec[0] >> 13
            he = hvec[1] >> 13
            kk = jnp.full((L,), k, jnp.int32)

            def hit_body(p, m):
                pk = pk_v[pl.ds(p, L)][0]
                ln = jnp.full((L,), pk & 511, jnp.int32)
                dest = pk >> 9
                for j in range(D // L):
                    v = plsc.load_gather(
                        slab_v, [kk, j * L + lane16, ln])
                    row_v[m, pl.ds(j * L, L)] = v
                plsc.store_scatter(dest_v, [jnp.full((L,), m, jnp.int32)],
                                   jnp.full((L,), dest, jnp.int32),
                                   mask=lane16 == 0)
                m = m + 1

                @pl.when(m == BATCH)
                def _():
                    fire_scatter()

                return jnp.where(m == BATCH, 0, m)

            return lax.fori_loop(hs, he, hit_body, m)

        m = lax.fori_loop(d_lo, d_hi, dd_body, 0)

        @pl.when(m > 0)
        def _():
            fire_scatter()

    @pl.when(c == 0)
    def _():
        run(e1_hbm, pk1_hbm, hpk1_hbm, g1_hbm, 0)

    @pl.when(c == 1)
    def _():
        run(e2_hbm, pk2_hbm, hpk2_hbm, g2_hbm, NS + 1)


def _p2_body(g1_hbm, g2_hbm, out_hbm, c1_v, c2_v, out_v):
    wid = lax.axis_index("s") * NC + lax.axis_index("c")
    base = wid * BPW
    lane16 = lax.iota(jnp.int32, L)

    for j in range(BPW // BATCH):
        pltpu.sync_copy(g1_hbm.at[pl.ds(base + j * BATCH, BATCH)], c1_v)
        pltpu.sync_copy(g2_hbm.at[pl.ds(base + j * BATCH, BATCH)], c2_v)

        def group_body(g, _):
            sums = jnp.zeros((L,), jnp.float32)
            for k in range(L):
                r = g * L + k
                acc = jnp.zeros((L,), jnp.float32)
                for q in range(D // L):
                    a = c1_v[r, pl.ds(q * L, L)]
                    b = c2_v[r, pl.ds(q * L, L)]
                    acc = acc + a * b
                sums = jnp.where(lane16 == k, jnp.sum(acc), sums)
            y = 1.0 / (1.0 + jnp.exp(-sums))
            out_v[pl.ds(j * BATCH + g * L, L)] = y
            return 0

        lax.fori_loop(0, BATCH // L, group_body, 0)

    pltpu.sync_copy(out_v, out_hbm.at[pl.ds(base, BPW)])


def _prep(idx):
    perm = jnp.argsort(idx).astype(jnp.int32)
    sr = idx[perm]
    keys = sr >> 9
    pos = jnp.arange(B, dtype=jnp.int32)
    flags = jnp.concatenate([jnp.ones((1,), jnp.bool_), keys[1:] != keys[:-1]])
    hstart = jnp.concatenate([jnp.sort(jnp.where(flags, pos, B)),
                              jnp.full((PAD_HPK - B,), B, jnp.int32)])
    keys_pad = jnp.concatenate([keys, jnp.full((1,), NSLAB, jnp.int32)])
    dlist = keys_pad[jnp.minimum(hstart, B)]
    base13 = jnp.minimum(dlist * (SS // LANES), MAXB13)
    hpk = (hstart << 13) | base13
    lane9 = sr - jnp.minimum((sr >> 9) * SS, MAXB13 * LANES)
    pk = jnp.concatenate([(perm << 9) | lane9,
                          jnp.zeros((PAD_HPK - B,), jnp.int32)])
    bnd = jnp.minimum(jnp.arange(NS + 1, dtype=jnp.int32) * SPT, NSLAB)
    td = jnp.searchsorted(dlist[:B], bnd).astype(jnp.int32)
    return pk, hpk, td


@jax.jit
def _run(X, emb1, emb2):
    wcf = X[:, 0]
    wof = X[:, 1]
    pk1, hpk1, td1 = _prep(wcf)
    pk2, hpk2, td2 = _prep(wof)
    ts = jnp.concatenate([td1, td2,
                          jnp.zeros((64 - 2 * (NS + 1),), jnp.int32)])

    mesh = plsc.VectorSubcoreMesh(core_axis_name="c", subcore_axis_name="s",
                                  num_cores=NC, num_subcores=NS)
    g1, g2 = pl.kernel(
        _p1_body,
        out_type=(jax.ShapeDtypeStruct((B + 1, LANES), jnp.float32),
                  jax.ShapeDtypeStruct((B + 1, LANES), jnp.float32)),
        mesh=mesh,
        scratch_types=[
            pltpu.VMEM((PAD_HPK,), jnp.int32),
            pltpu.VMEM((PAD_HPK,), jnp.int32),
            pltpu.VMEM((64,), jnp.int32),
            pltpu.VMEM((2, D, SS), jnp.float32),
            pltpu.VMEM((BATCH, LANES), jnp.float32),
            pltpu.VMEM((BATCH,), jnp.int32),
            pltpu.SemaphoreType.DMA,
            pltpu.SemaphoreType.DMA,
            pltpu.SemaphoreType.DMA,
        ],
        compiler_params=pltpu.CompilerParams(needs_layout_passes=False),
    )(emb1.T, emb2.T, pk1, pk2, hpk1, hpk2, ts)

    return pl.kernel(
        _p2_body,
        out_type=jax.ShapeDtypeStruct((B,), jnp.float32),
        mesh=mesh,
        scratch_types=[
            pltpu.VMEM((BATCH, LANES), jnp.float32),
            pltpu.VMEM((BATCH, LANES), jnp.float32),
            pltpu.VMEM((BPW,), jnp.float32),
        ],
        compiler_params=pltpu.CompilerParams(needs_layout_passes=False),
    )(g1, g2)


def kernel(X, emb1, emb2):
    return _run(X, emb1, emb2)


# speedup vs baseline: 5.6932x; 1.3256x over previous
import jax
import jax.numpy as jnp
from jax import lax
from jax.experimental import pallas as pl
from jax.experimental.pallas import tpu as pltpu
from jax.experimental.pallas import tpu_sc as plsc

NC = 2
NS = 16
L = 16
NW = NC * NS

B = 16384
V = 1000000
D = 64
LANES = 128
SS = 512
NSLAB = (V + SS - 1) // SS
SPT = (NSLAB + NS - 1) // NS
MAXB13 = (1000064 - SS) // LANES
BATCH = 128
DUMP = B
BPW = B // NW
PAD_HPK = 16400


def _p1_body(e1_hbm, e2_hbm, pk1_hbm, pk2_hbm, hpk1_hbm, hpk2_hbm, ts_hbm,
             g1_hbm, g2_hbm,
             pk_v, hpk_v, ts_v, slab_v, row_v, dest_v,
             sem_a, sem_b, sem_s):
    c = lax.axis_index("c")
    s = lax.axis_index("s")

    pltpu.sync_copy(ts_hbm, ts_v)

    def dest_reset():
        for k in range(BATCH // L):
            dest_v[pl.ds(k * L, L)] = jnp.full((L,), DUMP, jnp.int32)

    lane16 = lax.iota(jnp.int32, L)

    def run(e_hbm, pk_hbm, hpk_hbm, g_hbm, toff):
        pltpu.sync_copy(pk_hbm, pk_v)
        pltpu.sync_copy(hpk_hbm, hpk_v)
        dest_reset()

        tvec = ts_v[pl.ds(toff + s, L)]
        d_lo = tvec[0]
        d_hi = tvec[1]

        def slab_dma(dd, buf, sem):
            base13 = hpk_v[pl.ds(dd, L)][0] & 8191
            return pltpu.async_copy(
                e_hbm.at[:, pl.ds(base13 * LANES, SS)],
                slab_v.at[buf], sem)

        @pl.when(d_lo < d_hi)
        def _():
            slab_dma(d_lo, 0, sem_a)

        def fire_scatter():
            pltpu.async_copy(row_v, g_hbm.at[dest_v], sem_s).wait()
            dest_reset()

        def dd_body(dd, m):
            k = (dd - d_lo) % 2

            @pl.when((k == 0) & (dd + 1 < d_hi))
            def _():
                slab_dma(dd + 1, 1, sem_b)

            @pl.when((k == 1) & (dd + 1 < d_hi))
            def _():
                slab_dma(dd + 1, 0, sem_a)

            @pl.when(k == 0)
            def _():
                pltpu.make_async_copy(
                    e_hbm.at[:, pl.ds(0, SS)], slab_v.at[0], sem_a).wait()

            @pl.when(k == 1)
            def _():
                pltpu.make_async_copy(
                    e_hbm.at[:, pl.ds(0, SS)], slab_v.at[1], sem_b).wait()

            hvec = hpk_v[pl.ds(dd, L)]
            hs = hvec[0] >> 13
            he = hvec[1] >> 13
            kk = jnp.full((L,), k, jnp.int32)

            def hit_body(p, m):
                pk = pk_v[pl.ds(p, L)][0]
                ln = jnp.full((L,), pk & 511, jnp.int32)
                dest = pk >> 9
                for j in range(D // L):
                    v = plsc.load_gather(
                        slab_v, [kk, j * L + lane16, ln])
                    row_v[m, pl.ds(j * L, L)] = v
                plsc.store_scatter(dest_v, [jnp.full((L,), m, jnp.int32)],
                                   jnp.full((L,), dest, jnp.int32),
                                   mask=lane16 == 0)
                m = m + 1

                @pl.when(m == BATCH)
                def _():
                    fire_scatter()

                return jnp.where(m == BATCH, 0, m)

            return lax.fori_loop(hs, he, hit_body, m)

        m = lax.fori_loop(d_lo, d_hi, dd_body, 0)

        @pl.when(m > 0)
        def _():
            fire_scatter()

    @pl.when(c == 0)
    def _():
        run(e1_hbm, pk1_hbm, hpk1_hbm, g1_hbm, 0)

    @pl.when(c == 1)
    def _():
        run(e2_hbm, pk2_hbm, hpk2_hbm, g2_hbm, NS + 1)


def _p2_body(g1_hbm, g2_hbm, out_hbm, c1_v, c2_v, out_v):
    wid = lax.axis_index("s") * NC + lax.axis_index("c")
    base = wid * BPW
    lane16 = lax.iota(jnp.int32, L)

    for j in range(BPW // BATCH):
        pltpu.sync_copy(g1_hbm.at[pl.ds(base + j * BATCH, BATCH)], c1_v)
        pltpu.sync_copy(g2_hbm.at[pl.ds(base + j * BATCH, BATCH)], c2_v)

        def group_body(g, _):
            sums = jnp.zeros((L,), jnp.float32)
            for k in range(L):
                r = g * L + k
                acc = jnp.zeros((L,), jnp.float32)
                for q in range(D // L):
                    a = c1_v[r, pl.ds(q * L, L)]
                    b = c2_v[r, pl.ds(q * L, L)]
                    acc = acc + a * b
                sums = jnp.where(lane16 == k, jnp.sum(acc), sums)
            y = 1.0 / (1.0 + jnp.exp(-sums))
            out_v[pl.ds(j * BATCH + g * L, L)] = y
            return 0

        lax.fori_loop(0, BATCH // L, group_body, 0)

    pltpu.sync_copy(out_v, out_hbm.at[pl.ds(base, BPW)])


def _prep(idx):
    perm = jnp.argsort(idx).astype(jnp.int32)
    sr = idx[perm]
    keys = sr >> 9
    pos = jnp.arange(B, dtype=jnp.int32)
    flags = jnp.concatenate([jnp.ones((1,), jnp.bool_), keys[1:] != keys[:-1]])
    hstart = jnp.concatenate([jnp.sort(jnp.where(flags, pos, B)),
                              jnp.full((PAD_HPK - B,), B, jnp.int32)])
    dlist = jnp.concatenate([jnp.sort(jnp.where(flags, keys, NSLAB)),
                             jnp.full((PAD_HPK - B,), NSLAB, jnp.int32)])
    base13 = jnp.minimum(dlist * (SS // LANES), MAXB13)
    hpk = (hstart << 13) | base13
    lane9 = sr - jnp.minimum((sr >> 9) * SS, MAXB13 * LANES)
    pk = jnp.concatenate([(perm << 9) | lane9,
                          jnp.zeros((PAD_HPK - B,), jnp.int32)])
    bnd = jnp.minimum(jnp.arange(NS + 1, dtype=jnp.int32) * SPT, NSLAB)
    td = jnp.searchsorted(dlist[:B], bnd).astype(jnp.int32)
    return pk, hpk, td


@jax.jit
def _run(X, emb1, emb2):
    wcf = X[:, 0]
    wof = X[:, 1]
    pk1, hpk1, td1 = _prep(wcf)
    pk2, hpk2, td2 = _prep(wof)
    ts = jnp.concatenate([td1, td2,
                          jnp.zeros((64 - 2 * (NS + 1),), jnp.int32)])

    mesh = plsc.VectorSubcoreMesh(core_axis_name="c", subcore_axis_name="s",
                                  num_cores=NC, num_subcores=NS)
    g1, g2 = pl.kernel(
        _p1_body,
        out_type=(jax.ShapeDtypeStruct((B + 1, LANES), jnp.float32),
                  jax.ShapeDtypeStruct((B + 1, LANES), jnp.float32)),
        mesh=mesh,
        scratch_types=[
            pltpu.VMEM((PAD_HPK,), jnp.int32),
            pltpu.VMEM((PAD_HPK,), jnp.int32),
            pltpu.VMEM((64,), jnp.int32),
            pltpu.VMEM((2, D, SS), jnp.float32),
            pltpu.VMEM((BATCH, LANES), jnp.float32),
            pltpu.VMEM((BATCH,), jnp.int32),
            pltpu.SemaphoreType.DMA,
            pltpu.SemaphoreType.DMA,
            pltpu.SemaphoreType.DMA,
        ],
        compiler_params=pltpu.CompilerParams(needs_layout_passes=False),
    )(emb1.T, emb2.T, pk1, pk2, hpk1, hpk2, ts)

    return pl.kernel(
        _p2_body,
        out_type=jax.ShapeDtypeStruct((B,), jnp.float32),
        mesh=mesh,
        scratch_types=[
            pltpu.VMEM((BATCH, LANES), jnp.float32),
            pltpu.VMEM((BATCH, LANES), jnp.float32),
            pltpu.VMEM((BPW,), jnp.float32),
        ],
        compiler_params=pltpu.CompilerParams(needs_layout_passes=False),
    )(g1, g2)


def kernel(X, emb1, emb2):
    return _run(X, emb1, emb2)


# speedup vs baseline: 5.9719x; 1.0489x over previous
import jax
import jax.numpy as jnp
from jax import lax
from jax.experimental import pallas as pl
from jax.experimental.pallas import tpu as pltpu
from jax.experimental.pallas import tpu_sc as plsc

NC = 2
NS = 16
L = 16
NW = NC * NS

B = 16384
V = 1000000
D = 64
LANES = 128
SS = 256
NSLAB = (V + SS - 1) // SS
SPT = (NSLAB + NS - 1) // NS
MAXB13 = (1000064 - SS) // LANES
BATCH = 128
DUMP = B
BPW = B // NW
PAD_HPK = 16400


def _p1_body(e1_hbm, e2_hbm, pk1_hbm, pk2_hbm, hpk1_hbm, hpk2_hbm, ts_hbm,
             g1_hbm, g2_hbm,
             pk_v, hpk_v, ts_v, slab_v, row_v, dest_v,
             sem_a, sem_b, sem_c, sem_s):
    c = lax.axis_index("c")
    s = lax.axis_index("s")

    pltpu.sync_copy(ts_hbm, ts_v)

    def dest_reset():
        for k in range(BATCH // L):
            dest_v[pl.ds(k * L, L)] = jnp.full((L,), DUMP, jnp.int32)

    lane16 = lax.iota(jnp.int32, L)

    def run(e_hbm, pk_hbm, hpk_hbm, g_hbm, toff):
        pltpu.sync_copy(pk_hbm, pk_v)
        pltpu.sync_copy(hpk_hbm, hpk_v)
        dest_reset()

        tvec = ts_v[pl.ds(toff + s, L)]
        d_lo = tvec[0]
        d_hi = tvec[1]

        def slab_dma(dd, buf, sem):
            base13 = hpk_v[pl.ds(dd, L)][0] & 8191
            return pltpu.async_copy(
                e_hbm.at[:, pl.ds(base13 * LANES, SS)],
                slab_v.at[buf], sem)

        @pl.when(d_lo < d_hi)
        def _():
            slab_dma(d_lo, 0, sem_a)

        @pl.when(d_lo + 1 < d_hi)
        def _():
            slab_dma(d_lo + 1, 1, sem_b)

        def fire_scatter():
            pltpu.async_copy(row_v, g_hbm.at[dest_v], sem_s).wait()
            dest_reset()

        def dd_body(dd, m):
            k = (dd - d_lo) % 3

            @pl.when((k == 0) & (dd + 2 < d_hi))
            def _():
                slab_dma(dd + 2, 2, sem_c)

            @pl.when((k == 1) & (dd + 2 < d_hi))
            def _():
                slab_dma(dd + 2, 0, sem_a)

            @pl.when((k == 2) & (dd + 2 < d_hi))
            def _():
                slab_dma(dd + 2, 1, sem_b)

            @pl.when(k == 0)
            def _():
                pltpu.make_async_copy(
                    e_hbm.at[:, pl.ds(0, SS)], slab_v.at[0], sem_a).wait()

            @pl.when(k == 1)
            def _():
                pltpu.make_async_copy(
                    e_hbm.at[:, pl.ds(0, SS)], slab_v.at[1], sem_b).wait()

            @pl.when(k == 2)
            def _():
                pltpu.make_async_copy(
                    e_hbm.at[:, pl.ds(0, SS)], slab_v.at[2], sem_c).wait()

            hvec = hpk_v[pl.ds(dd, L)]
            hs = hvec[0] >> 13
            he = hvec[1] >> 13
            kk = jnp.full((L,), k, jnp.int32)

            def hit_body(p, m):
                pk = pk_v[pl.ds(p, L)][0]
                ln = jnp.full((L,), pk & 511, jnp.int32)
                dest = pk >> 9
                for j in range(D // L):
                    v = plsc.load_gather(
                        slab_v, [kk, j * L + lane16, ln])
                    row_v[m, pl.ds(j * L, L)] = v
                plsc.store_scatter(dest_v, [jnp.full((L,), m, jnp.int32)],
                                   jnp.full((L,), dest, jnp.int32),
                                   mask=lane16 == 0)
                m = m + 1

                @pl.when(m == BATCH)
                def _():
                    fire_scatter()

                return jnp.where(m == BATCH, 0, m)

            return lax.fori_loop(hs, he, hit_body, m)

        m = lax.fori_loop(d_lo, d_hi, dd_body, 0)

        @pl.when(m > 0)
        def _():
            fire_scatter()

    @pl.when(c == 0)
    def _():
        run(e1_hbm, pk1_hbm, hpk1_hbm, g1_hbm, 0)

    @pl.when(c == 1)
    def _():
        run(e2_hbm, pk2_hbm, hpk2_hbm, g2_hbm, NS + 1)


def _p2_body(g1_hbm, g2_hbm, out_hbm, c1_v, c2_v, out_v):
    wid = lax.axis_index("s") * NC + lax.axis_index("c")
    base = wid * BPW
    lane16 = lax.iota(jnp.int32, L)

    for j in range(BPW // BATCH):
        pltpu.sync_copy(g1_hbm.at[pl.ds(base + j * BATCH, BATCH)], c1_v)
        pltpu.sync_copy(g2_hbm.at[pl.ds(base + j * BATCH, BATCH)], c2_v)

        def group_body(g, _):
            sums = jnp.zeros((L,), jnp.float32)
            for k in range(L):
                r = g * L + k
                acc = jnp.zeros((L,), jnp.float32)
                for q in range(D // L):
                    a = c1_v[r, pl.ds(q * L, L)]
                    b = c2_v[r, pl.ds(q * L, L)]
                    acc = acc + a * b
                sums = jnp.where(lane16 == k, jnp.sum(acc), sums)
            y = 1.0 / (1.0 + jnp.exp(-sums))
            out_v[pl.ds(j * BATCH + g * L, L)] = y
            return 0

        lax.fori_loop(0, BATCH // L, group_body, 0)

    pltpu.sync_copy(out_v, out_hbm.at[pl.ds(base, BPW)])


def _prep(idx):
    perm = jnp.argsort(idx).astype(jnp.int32)
    sr = idx[perm]
    keys = sr // SS
    pos = jnp.arange(B, dtype=jnp.int32)
    flags = jnp.concatenate([jnp.ones((1,), jnp.bool_), keys[1:] != keys[:-1]])
    hstart = jnp.concatenate([jnp.sort(jnp.where(flags, pos, B)),
                              jnp.full((PAD_HPK - B,), B, jnp.int32)])
    dlist = jnp.concatenate([jnp.sort(jnp.where(flags, keys, NSLAB)),
                             jnp.full((PAD_HPK - B,), NSLAB, jnp.int32)])
    base13 = jnp.minimum(dlist * (SS // LANES), MAXB13)
    hpk = (hstart << 13) | base13
    lane9 = sr - jnp.minimum((sr // SS) * SS, MAXB13 * LANES)
    pk = jnp.concatenate([(perm << 9) | lane9,
                          jnp.zeros((PAD_HPK - B,), jnp.int32)])
    bnd = jnp.minimum(jnp.arange(NS + 1, dtype=jnp.int32) * SPT, NSLAB)
    td = jnp.searchsorted(dlist[:B], bnd).astype(jnp.int32)
    return pk, hpk, td


@jax.jit
def _run(X, emb1, emb2):
    wcf = X[:, 0]
    wof = X[:, 1]
    pk1, hpk1, td1 = _prep(wcf)
    pk2, hpk2, td2 = _prep(wof)
    ts = jnp.concatenate([td1, td2,
                          jnp.zeros((64 - 2 * (NS + 1),), jnp.int32)])

    mesh = plsc.VectorSubcoreMesh(core_axis_name="c", subcore_axis_name="s",
                                  num_cores=NC, num_subcores=NS)
    g1, g2 = pl.kernel(
        _p1_body,
        out_type=(jax.ShapeDtypeStruct((B + 1, LANES), jnp.float32),
                  jax.ShapeDtypeStruct((B + 1, LANES), jnp.float32)),
        mesh=mesh,
        scratch_types=[
            pltpu.VMEM((PAD_HPK,), jnp.int32),
            pltpu.VMEM((PAD_HPK,), jnp.int32),
            pltpu.VMEM((64,), jnp.int32),
            pltpu.VMEM((3, D, SS), jnp.float32),
            pltpu.VMEM((BATCH, LANES), jnp.float32),
            pltpu.VMEM((BATCH,), jnp.int32),
            pltpu.SemaphoreType.DMA,
            pltpu.SemaphoreType.DMA,
            pltpu.SemaphoreType.DMA,
            pltpu.SemaphoreType.DMA,
        ],
        compiler_params=pltpu.CompilerParams(needs_layout_passes=False),
    )(emb1.T, emb2.T, pk1, pk2, hpk1, hpk2, ts)

    return pl.kernel(
        _p2_body,
        out_type=jax.ShapeDtypeStruct((B,), jnp.float32),
        mesh=mesh,
        scratch_types=[
            pltpu.VMEM((BATCH, LANES), jnp.float32),
            pltpu.VMEM((BATCH, LANES), jnp.float32),
            pltpu.VMEM((BPW,), jnp.float32),
        ],
        compiler_params=pltpu.CompilerParams(needs_layout_passes=False),
    )(g1, g2)


def kernel(X, emb1, emb2):
    return _run(X, emb1, emb2)


# speedup vs baseline: 6.6469x; 1.1130x over previous
import jax
import jax.numpy as jnp
from jax import lax
from jax.experimental import pallas as pl
from jax.experimental.pallas import tpu as pltpu
from jax.experimental.pallas import tpu_sc as plsc

NC = 2
NS = 16
L = 16
NW = NC * NS

B = 16384
V = 1000000
D = 64
LANES = 128
SS = 256
NSLAB = (V + SS - 1) // SS
SPT = (NSLAB + NS - 1) // NS
MAXB13 = (1000064 - SS) // LANES
BATCH = 128
DUMP = B
BPW = B // NW
PAD_HPK = 16400


def _p1_body(e1_hbm, e2_hbm, pk1_hbm, pk2_hbm, hpk1_hbm, hpk2_hbm, ts_hbm,
             g1_hbm, g2_hbm,
             pk_v, hpk_v, ts_v, slab_v, row_v, dest_v,
             sem_a, sem_b, sem_c, sem_d, sem_s):
    c = lax.axis_index("c")
    s = lax.axis_index("s")

    pltpu.sync_copy(ts_hbm, ts_v)

    def dest_reset():
        for k in range(BATCH // L):
            dest_v[pl.ds(k * L, L)] = jnp.full((L,), DUMP, jnp.int32)

    lane16 = lax.iota(jnp.int32, L)

    def run(e_hbm, pk_hbm, hpk_hbm, g_hbm, toff):
        pltpu.sync_copy(pk_hbm, pk_v)
        pltpu.sync_copy(hpk_hbm, hpk_v)
        dest_reset()

        tvec = ts_v[pl.ds(toff + s, L)]
        d_lo = tvec[0]
        d_hi = tvec[1]

        def slab_dma(dd, buf, sem):
            base13 = hpk_v[pl.ds(dd, L)][0] & 8191
            return pltpu.async_copy(
                e_hbm.at[:, pl.ds(base13 * LANES, SS)],
                slab_v.at[buf], sem)

        @pl.when(d_lo < d_hi)
        def _():
            slab_dma(d_lo, 0, sem_a)

        @pl.when(d_lo + 1 < d_hi)
        def _():
            slab_dma(d_lo + 1, 1, sem_b)

        @pl.when(d_lo + 2 < d_hi)
        def _():
            slab_dma(d_lo + 2, 2, sem_c)

        def fire_scatter():
            pltpu.async_copy(row_v, g_hbm.at[dest_v], sem_s).wait()
            dest_reset()

        def dd_body(dd, m):
            k = (dd - d_lo) % 4

            @pl.when((k == 0) & (dd + 3 < d_hi))
            def _():
                slab_dma(dd + 3, 3, sem_d)

            @pl.when((k == 1) & (dd + 3 < d_hi))
            def _():
                slab_dma(dd + 3, 0, sem_a)

            @pl.when((k == 2) & (dd + 3 < d_hi))
            def _():
                slab_dma(dd + 3, 1, sem_b)

            @pl.when((k == 3) & (dd + 3 < d_hi))
            def _():
                slab_dma(dd + 3, 2, sem_c)

            @pl.when(k == 0)
            def _():
                pltpu.make_async_copy(
                    e_hbm.at[:, pl.ds(0, SS)], slab_v.at[0], sem_a).wait()

            @pl.when(k == 1)
            def _():
                pltpu.make_async_copy(
                    e_hbm.at[:, pl.ds(0, SS)], slab_v.at[1], sem_b).wait()

            @pl.when(k == 2)
            def _():
                pltpu.make_async_copy(
                    e_hbm.at[:, pl.ds(0, SS)], slab_v.at[2], sem_c).wait()

            @pl.when(k == 3)
            def _():
                pltpu.make_async_copy(
                    e_hbm.at[:, pl.ds(0, SS)], slab_v.at[3], sem_d).wait()

            hvec = hpk_v[pl.ds(dd, L)]
            hs = hvec[0] >> 13
            he = hvec[1] >> 13
            kk = jnp.full((L,), k, jnp.int32)

            def hit_body(p, m):
                pk = pk_v[pl.ds(p, L)][0]
                ln = jnp.full((L,), pk & 511, jnp.int32)
                dest = pk >> 9
                for j in range(D // L):
                    v = plsc.load_gather(
                        slab_v, [kk, j * L + lane16, ln])
                    row_v[m, pl.ds(j * L, L)] = v
                plsc.store_scatter(dest_v, [jnp.full((L,), m, jnp.int32)],
                                   jnp.full((L,), dest, jnp.int32),
                                   mask=lane16 == 0)
                m = m + 1

                @pl.when(m == BATCH)
                def _():
                    fire_scatter()

                return jnp.where(m == BATCH, 0, m)

            return lax.fori_loop(hs, he, hit_body, m)

        m = lax.fori_loop(d_lo, d_hi, dd_body, 0)

        @pl.when(m > 0)
        def _():
            fire_scatter()

    @pl.when(c == 0)
    def _():
        run(e1_hbm, pk1_hbm, hpk1_hbm, g1_hbm, 0)

    @pl.when(c == 1)
    def _():
        run(e2_hbm, pk2_hbm, hpk2_hbm, g2_hbm, NS + 1)


def _p2_body(g1_hbm, g2_hbm, out_hbm, c1_v, c2_v, out_v):
    wid = lax.axis_index("s") * NC + lax.axis_index("c")
    base = wid * BPW
    lane16 = lax.iota(jnp.int32, L)

    for j in range(BPW // BATCH):
        pltpu.sync_copy(g1_hbm.at[pl.ds(base + j * BATCH, BATCH)], c1_v)
        pltpu.sync_copy(g2_hbm.at[pl.ds(base + j * BATCH, BATCH)], c2_v)

        def group_body(g, _):
            sums = jnp.zeros((L,), jnp.float32)
            for k in range(L):
                r = g * L + k
                acc = jnp.zeros((L,), jnp.float32)
                for q in range(D // L):
                    a = c1_v[r, pl.ds(q * L, L)]
                    b = c2_v[r, pl.ds(q * L, L)]
                    acc = acc + a * b
                sums = jnp.where(lane16 == k, jnp.sum(acc), sums)
            y = 1.0 / (1.0 + jnp.exp(-sums))
            out_v[pl.ds(j * BATCH + g * L, L)] = y
            return 0

        lax.fori_loop(0, BATCH // L, group_body, 0)

    pltpu.sync_copy(out_v, out_hbm.at[pl.ds(base, BPW)])


def _prep(idx):
    perm = jnp.argsort(idx).astype(jnp.int32)
    sr = idx[perm]
    keys = sr // SS
    pos = jnp.arange(B, dtype=jnp.int32)
    flags = jnp.concatenate([jnp.ones((1,), jnp.bool_), keys[1:] != keys[:-1]])
    packed = jnp.sort(jnp.where(flags, (keys << 14) | pos,
                                (NSLAB << 14) | (B - 1)))
    packed = jnp.concatenate([packed,
                              jnp.full((PAD_HPK - B,),
                                       (NSLAB << 14) | (B - 1), jnp.int32)])
    dlist = packed >> 14
    hstart = jnp.where(dlist >= NSLAB, B, packed & (B - 1))
    base13 = jnp.minimum(dlist * (SS // LANES), MAXB13)
    hpk = (hstart << 13) | base13
    lane9 = sr - jnp.minimum((sr // SS) * SS, MAXB13 * LANES)
    pk = jnp.concatenate([(perm << 9) | lane9,
                          jnp.zeros((PAD_HPK - B,), jnp.int32)])
    bnd = jnp.minimum(jnp.arange(NS + 1, dtype=jnp.int32) * SPT, NSLAB)
    td = jnp.searchsorted(dlist[:B], bnd).astype(jnp.int32)
    return pk, hpk, td


@jax.jit
def _run(X, emb1, emb2):
    wcf = X[:, 0]
    wof = X[:, 1]
    pk1, hpk1, td1 = _prep(wcf)
    pk2, hpk2, td2 = _prep(wof)
    ts = jnp.concatenate([td1, td2,
                          jnp.zeros((64 - 2 * (NS + 1),), jnp.int32)])

    mesh = plsc.VectorSubcoreMesh(core_axis_name="c", subcore_axis_name="s",
                                  num_cores=NC, num_subcores=NS)
    g1, g2 = pl.kernel(
        _p1_body,
        out_type=(jax.ShapeDtypeStruct((B + 1, LANES), jnp.float32),
                  jax.ShapeDtypeStruct((B + 1, LANES), jnp.float32)),
        mesh=mesh,
        scratch_types=[
            pltpu.VMEM((PAD_HPK,), jnp.int32),
            pltpu.VMEM((PAD_HPK,), jnp.int32),
            pltpu.VMEM((64,), jnp.int32),
            pltpu.VMEM((4, D, SS), jnp.float32),
            pltpu.VMEM((BATCH, LANES), jnp.float32),
            pltpu.VMEM((BATCH,), jnp.int32),
            pltpu.SemaphoreType.DMA,
            pltpu.SemaphoreType.DMA,
            pltpu.SemaphoreType.DMA,
            pltpu.SemaphoreType.DMA,
            pltpu.SemaphoreType.DMA,
        ],
        compiler_params=pltpu.CompilerParams(needs_layout_passes=False),
    )(emb1.T, emb2.T, pk1, pk2, hpk1, hpk2, ts)

    return pl.kernel(
        _p2_body,
        out_type=jax.ShapeDtypeStruct((B,), jnp.float32),
        mesh=mesh,
        scratch_types=[
            pltpu.VMEM((BATCH, LANES), jnp.float32),
            pltpu.VMEM((BATCH, LANES), jnp.float32),
            pltpu.VMEM((BPW,), jnp.float32),
        ],
        compiler_params=pltpu.CompilerParams(needs_layout_passes=False),
    )(g1, g2)


def kernel(X, emb1, emb2):
    return _run(X, emb1, emb2)
